# baseline (speedup 1.0000x reference)
import jax
import jax.numpy as jnp
from jax import lax
from jax.experimental import pallas as pl
from jax.experimental.pallas import tpu as pltpu
from jax.experimental.pallas import tpu_sc as plsc

_B = 16384
_D = 32
_SUB = 8
_NTROW = 1000000 // _SUB
_NC = 2
_NS = 16
_NW = _NC * _NS
_BPW = _B // _NW
_LANES = 16
_NSTEP = _BPW // _LANES


def _gmf_body(tidx_u_hbm, tidx_i_hbm, sub_u_hbm, sub_i_hbm, utab_hbm,
              itab_hbm, wb_hbm, bias_hbm, out_hbm,
              tidx_u_v, tidx_i_v, sub_u_v, sub_i_v, urows_v, irows_v,
              wb_v, bias_v, out_v, sem0, sem1, sem2, sem3):
    sems = (sem0, sem1, sem2, sem3)
    wid = lax.axis_index("s") * _NC + lax.axis_index("c")
    base = wid * _BPW

    pltpu.sync_copy(tidx_u_hbm.at[pl.ds(base, _BPW)], tidx_u_v)
    pltpu.sync_copy(tidx_i_hbm.at[pl.ds(base, _BPW)], tidx_i_v)
    pltpu.sync_copy(sub_u_hbm.at[pl.ds(base, _BPW)], sub_u_v)
    pltpu.sync_copy(sub_i_hbm.at[pl.ds(base, _BPW)], sub_i_v)
    pltpu.sync_copy(wb_hbm, wb_v)
    pltpu.sync_copy(bias_hbm, bias_v)

    bias = bias_v[...]
    wcols = [wb_v[pl.ds(d * _LANES, _LANES)] for d in range(_D)]
    riota = lax.iota(jnp.int32, _LANES)

    def step(s, carry):
        off = s * _LANES
        tu = tidx_u_v[pl.ds(off, _LANES)]
        ti = tidx_i_v[pl.ds(off, _LANES)]
        copies = []
        for k in range(_LANES):
            ru = pl.multiple_of(tu[k] * _SUB, _SUB)
            ri = pl.multiple_of(ti[k] * _SUB, _SUB)
            copies.append(pltpu.async_copy(
                utab_hbm.at[pl.ds(ru, _SUB)], urows_v.at[k],
                sems[(2 * k) % 4]))
            copies.append(pltpu.async_copy(
                itab_hbm.at[pl.ds(ri, _SUB)], irows_v.at[k],
                sems[(2 * k + 1) % 4]))
        for c in copies:
            c.wait()
        su = sub_u_v[pl.ds(off, _LANES)]
        si = sub_i_v[pl.ds(off, _LANES)]
        acc = bias
        for d in range(_D):
            cd = jnp.full((_LANES,), d, jnp.int32)
            u = plsc.load_gather(urows_v, [riota, su, cd])
            it = plsc.load_gather(irows_v, [riota, si, cd])
            acc = acc + u * it * wcols[d]
        out_v[pl.ds(off, _LANES)] = acc
        return carry

    lax.fori_loop(0, _NSTEP, step, 0)

    pltpu.sync_copy(out_v, out_hbm.at[pl.ds(base, _BPW)])


def kernel(user_indices, item_indices, user_table, item_table, fc_w, fc_b):
    ui = user_indices.astype(jnp.int32)
    ii = item_indices.astype(jnp.int32)
    tidx_u = ui // _SUB
    tidx_i = ii // _SUB
    sub_u = ui % _SUB
    sub_i = ii % _SUB
    utab = user_table
    itab = item_table
    w = fc_w.reshape(_D).astype(jnp.float32)
    wb = jnp.broadcast_to(w[:, None], (_D, _LANES)).reshape(_D * _LANES)
    bias = jnp.broadcast_to(fc_b.reshape(()), (_LANES,)).astype(jnp.float32)

    run = pl.kernel(
        _gmf_body,
        out_type=jax.ShapeDtypeStruct((_B,), jnp.float32),
        mesh=plsc.VectorSubcoreMesh(
            core_axis_name="c", subcore_axis_name="s",
            num_cores=_NC, num_subcores=_NS),
        compiler_params=pltpu.CompilerParams(needs_layout_passes=False),
        scratch_types=[
            pltpu.VMEM((_BPW,), jnp.int32),
            pltpu.VMEM((_BPW,), jnp.int32),
            pltpu.VMEM((_BPW,), jnp.int32),
            pltpu.VMEM((_BPW,), jnp.int32),
            pltpu.VMEM((_LANES, _SUB, _D), jnp.float32),
            pltpu.VMEM((_LANES, _SUB, _D), jnp.float32),
            pltpu.VMEM((_D * _LANES,), jnp.float32),
            pltpu.VMEM((_LANES,), jnp.float32),
            pltpu.VMEM((_BPW,), jnp.float32),
            pltpu.SemaphoreType.DMA,
            pltpu.SemaphoreType.DMA,
            pltpu.SemaphoreType.DMA,
            pltpu.SemaphoreType.DMA,
        ],
    )
    out = run(tidx_u, tidx_i, sub_u, sub_i, utab, itab, wb, bias)
    return out.reshape(_B, 1)


# speedup vs baseline: 1.0349x; 1.0349x over previous
import jax
import jax.numpy as jnp
from jax import lax
from jax.experimental import pallas as pl
from jax.experimental.pallas import tpu as pltpu
from jax.experimental.pallas import tpu_sc as plsc

_B = 16384
_D = 32
_SUB = 8
_NTROW = 1000000 // _SUB
_NC = 2
_NS = 16
_NW = _NC * _NS
_BPW = _B // _NW
_LANES = 16
_NSTEP = _BPW // _LANES


def _gmf_body(tidx_u_hbm, tidx_i_hbm, utab_hbm,
              itab_hbm, wb_hbm, bias_hbm, out_hbm,
              tidx_u_v, tidx_i_v, urows_v, irows_v,
              wb_v, bias_v, out_v, sem0, sem1, sem2, sem3):
    sems = (sem0, sem1, sem2, sem3)
    wid = lax.axis_index("s") * _NC + lax.axis_index("c")
    base = wid * _BPW

    pltpu.sync_copy(tidx_u_hbm.at[pl.ds(base, _BPW)], tidx_u_v)
    pltpu.sync_copy(tidx_i_hbm.at[pl.ds(base, _BPW)], tidx_i_v)
    pltpu.sync_copy(wb_hbm, wb_v)
    pltpu.sync_copy(bias_hbm, bias_v)

    bias = bias_v[...]
    wcols = [wb_v[pl.ds(d * _LANES, _LANES)] for d in range(_D)]
    riota = lax.iota(jnp.int32, _LANES)

    def step(s, carry):
        off = s * _LANES
        tu = tidx_u_v[pl.ds(off, _LANES)]
        ti = tidx_i_v[pl.ds(off, _LANES)]
        for h in range(2):
            copies = []
            for k in range(h * 8, h * 8 + 8):
                copies.append(pltpu.async_copy(
                    utab_hbm.at[pl.ds(tu[k], 1)], urows_v.at[pl.ds(k, 1)],
                    sems[(2 * k) % 4]))
                copies.append(pltpu.async_copy(
                    itab_hbm.at[pl.ds(ti[k], 1)], irows_v.at[pl.ds(k, 1)],
                    sems[(2 * k + 1) % 4]))
            for c in copies:
                c.wait()
        acc = bias
        for d in range(_D):
            cd = jnp.full((_LANES,), d, jnp.int32)
            u = plsc.load_gather(urows_v, [riota, cd])
            it = plsc.load_gather(irows_v, [riota, cd])
            acc = acc + u * it * wcols[d]
        out_v[pl.ds(off, _LANES)] = acc
        return carry

    lax.fori_loop(0, _NSTEP, step, 0)

    pltpu.sync_copy(out_v, out_hbm.at[pl.ds(base, _BPW)])


def kernel(user_indices, item_indices, user_table, item_table, fc_w, fc_b):
    tidx_u = user_indices.astype(jnp.int32)
    tidx_i = item_indices.astype(jnp.int32)
    utab = user_table
    itab = item_table
    w = fc_w.reshape(_D).astype(jnp.float32)
    wb = jnp.broadcast_to(w[:, None], (_D, _LANES)).reshape(_D * _LANES)
    bias = jnp.broadcast_to(fc_b.reshape(()), (_LANES,)).astype(jnp.float32)

    run = pl.kernel(
        _gmf_body,
        out_type=jax.ShapeDtypeStruct((_B,), jnp.float32),
        mesh=plsc.VectorSubcoreMesh(
            core_axis_name="c", subcore_axis_name="s",
            num_cores=_NC, num_subcores=_NS),
        compiler_params=pltpu.CompilerParams(needs_layout_passes=False),
        scratch_types=[
            pltpu.VMEM((_BPW,), jnp.int32),
            pltpu.VMEM((_BPW,), jnp.int32),
            pltpu.VMEM((_LANES, _D), jnp.float32),
            pltpu.VMEM((_LANES, _D), jnp.float32),
            pltpu.VMEM((_D * _LANES,), jnp.float32),
            pltpu.VMEM((_LANES,), jnp.float32),
            pltpu.VMEM((_BPW,), jnp.float32),
            pltpu.SemaphoreType.DMA,
            pltpu.SemaphoreType.DMA,
            pltpu.SemaphoreType.DMA,
            pltpu.SemaphoreType.DMA,
        ],
    )
    out = run(tidx_u, tidx_i, utab, itab, wb, bias)
    return out.reshape(_B, 1)
